# Initial kernel scaffold; baseline (speedup 1.0000x reference)
import jax
import jax.numpy as jnp
from jax.experimental import pallas as pl


def kernel(entity, edge_index, edge_type, edge_norm, emb, W1_rel, W1_root, W2_rel, W2_root):
    raise NotImplementedError("write your pallas kernel here")


# speedup vs baseline: 5.6635x; 5.6635x over previous
import functools

import jax
import jax.numpy as jnp
from jax import lax
from jax.experimental import pallas as pl
from jax.experimental.pallas import tpu as pltpu
from jax.experimental.pallas import tpu_sc as plsc

N = 50000
E = 800000
D = 100
R = 16
DP = 128
CW = 32
NCH = 4
EP = 819200
EROWS = EP // 128
TB = 400
ACC = 51200
PT = ACC // 16
ENT_PAD = 53248
NB = 1000
CROWS = EROWS // 2
CPT = CROWS // 16
K = 4

_sc_mesh = plsc.VectorSubcoreMesh(core_axis_name="c", subcore_axis_name="s")


@functools.partial(
    pl.kernel,
    out_type=jax.ShapeDtypeStruct((ENT_PAD, DP), jnp.float32),
    mesh=_sc_mesh,
    scratch_types=[
        pltpu.VMEM((128,), jnp.int32),
        pltpu.VMEM((128, DP), jnp.float32),
        pltpu.SemaphoreType.DMA,
    ],
)
def _gather_rows(table_hbm, idx_hbm, out_hbm, idx_v, rows_v, sem):
    wid = lax.axis_index("s") * 2 + lax.axis_index("c")
    base = wid * (ENT_PAD // 32)
    for b in range(ENT_PAD // 32 // 128):
        off = base + b * 128
        pltpu.sync_copy(idx_hbm.at[pl.ds(off, 128)], idx_v)
        pltpu.async_copy(table_hbm.at[idx_v], rows_v, sem).wait()
        pltpu.sync_copy(rows_v, out_hbm.at[pl.ds(off, 128), :])


def _make_agg(do_count: bool):
    chunk_out = jax.ShapeDtypeStruct((NCH, ACC, CW), jnp.float32)
    cnt_out = jax.ShapeDtypeStruct((2, ACC, CW), jnp.float32)
    out_type = (cnt_out, chunk_out) if do_count else chunk_out
    scratch = [
        pltpu.VMEM_SHARED((ACC, CW), jnp.float32),
        pltpu.VMEM((K, 128), jnp.int32),
        pltpu.VMEM((K, 128), jnp.int32),
        pltpu.VMEM((8, 128), jnp.int32),
        pltpu.VMEM((K, 128, CW), jnp.float32),
        pltpu.VMEM((64, CW), jnp.float32),
        pltpu.VMEM((128, CW), jnp.float32),
        pltpu.SemaphoreType.DMA,
        pltpu.SemaphoreType.DMA,
    ]

    def body(z_hbm, ridx_hbm, dst_hbm, *refs):
        if do_count:
            cnt_hbm, out_hbm = refs[0], refs[1]
            scr = refs[2:]
        else:
            out_hbm = refs[0]
            scr = refs[1:]
        acc, ibuf, dbuf, cbuf, rows, zbuf, obuf, gsem, ssem = scr
        core = lax.axis_index("c")
        sub = lax.axis_index("s")

        zeros16 = jnp.zeros((16,), jnp.float32)
        ones16 = jnp.ones((16,), jnp.float32)

        def fillz(i, _):
            zbuf[i // 2, pl.ds((i % 2) * 16, 16)] = zeros16
            return 0
        lax.fori_loop(0, 128, fillz, 0)

        def fillo(i, _):
            obuf[i // 2, pl.ds((i % 2) * 16, 16)] = ones16
            return 0
        lax.fori_loop(0, 256, fillo, 0)

        my_acc0 = sub * PT

        def zero_acc():
            for z in range(PT // 64):
                pltpu.sync_copy(zbuf, acc.at[pl.ds(my_acc0 + z * 64, 64), :])

        def drain(dst_ref):
            pltpu.sync_copy(acc.at[pl.ds(my_acc0, PT), :],
                            dst_ref.at[pl.ds(my_acc0, PT), :])

        if do_count:
            zero_acc()
            plsc.subcore_barrier()
            cbase = core * CROWS + sub * CPT

            def cgroup(g, _):
                pltpu.sync_copy(dst_hbm.at[pl.ds(cbase + g * 8, 8), :], cbuf)
                hs = [pltpu.async_copy(obuf, acc.at[cbuf.at[j]], ssem, add=True)
                      for j in range(8)]
                for h in hs:
                    h.wait()
                return 0
            lax.fori_loop(0, CPT // 8, cgroup, 0)
            plsc.subcore_barrier()
            drain(cnt_hbm.at[core])

        for j in range(2):
            chunk = 2 * core + j
            plsc.subcore_barrier()
            zero_acc()
            plsc.subcore_barrier()
            ibase = chunk * EROWS + sub * TB
            dbase = sub * TB

            def group(g, _):
                pltpu.sync_copy(ridx_hbm.at[pl.ds(ibase + g * K, K), :], ibuf)
                pltpu.sync_copy(dst_hbm.at[pl.ds(dbase + g * K, K), :], dbuf)
                ghs = [pltpu.async_copy(z_hbm.at[ibuf.at[k]], rows.at[k], gsem)
                       for k in range(K)]
                for h in ghs:
                    h.wait()
                shs = [pltpu.async_copy(rows.at[k], acc.at[dbuf.at[k]], ssem,
                                        add=True)
                       for k in range(K)]
                for h in shs:
                    h.wait()
                return 0
            lax.fori_loop(0, TB // K, group, 0)
            plsc.subcore_barrier()
            drain(out_hbm.at[chunk])

    return pl.kernel(body, out_type=out_type, mesh=_sc_mesh,
                     scratch_types=scratch,
                     compiler_params=pltpu.CompilerParams(
                         use_tc_tiling_on_sc=False))


_agg_l1 = _make_agg(True)
_agg_l2 = _make_agg(False)


def _prep_ridx(et2d, src2d):
    def body(t_ref, s_ref, o_ref):
        c = pl.program_id(0)
        o_ref[...] = (t_ref[...] * N + s_ref[...]) * NCH + c

    return pl.pallas_call(
        body,
        grid=(NCH,),
        in_specs=[
            pl.BlockSpec((EROWS, 128), lambda c: (0, 0)),
            pl.BlockSpec((EROWS, 128), lambda c: (0, 0)),
        ],
        out_specs=pl.BlockSpec((EROWS, 128), lambda c: (c, 0)),
        out_shape=jax.ShapeDtypeStruct((NCH * EROWS, 128), jnp.int32),
    )(et2d, src2d)


def _rel_transform(x, w_pad):
    nblk = N // NB

    def body(x_ref, w_ref, o_ref):
        o_ref[...] = lax.dot_general(x_ref[...], w_ref[0],
                                     (((1,), (1,)), ((), ())),
                                     preferred_element_type=jnp.float32)

    return pl.pallas_call(
        body,
        grid=(nblk, R),
        in_specs=[
            pl.BlockSpec((NB, DP), lambda nb, r: (nb, 0)),
            pl.BlockSpec((1, DP, DP), lambda nb, r: (r, 0, 0)),
        ],
        out_specs=pl.BlockSpec((NB, DP), lambda nb, r: (r * nblk + nb, 0)),
        out_shape=jax.ShapeDtypeStruct((R * N, DP), jnp.float32),
    )(x, w_pad)


def _combine(cnt_acc, ch_acc, x, w_root_pad, relu, dout):
    def body(c_ref, a_ref, x_ref, w_ref, o_ref):
        cat = jnp.concatenate([a_ref[c] for c in range(NCH)], axis=1)
        cnt = c_ref[0][:, 0:1] + c_ref[1][:, 0:1]
        inv = 1.0 / jnp.maximum(cnt, 1.0)
        root = lax.dot_general(x_ref[...], w_ref[...], (((1,), (1,)), ((), ())),
                               preferred_element_type=jnp.float32)
        y = cat * inv + root
        if relu:
            y = jnp.maximum(y, 0.0)
        o_ref[...] = y[:, :dout]

    return pl.pallas_call(
        body,
        grid=(N // NB,),
        in_specs=[
            pl.BlockSpec((2, NB, CW), lambda nb: (0, nb, 0)),
            pl.BlockSpec((NCH, NB, CW), lambda nb: (0, nb, 0)),
            pl.BlockSpec((NB, DP), lambda nb: (nb, 0)),
            pl.BlockSpec((DP, DP), lambda nb: (0, 0)),
        ],
        out_specs=pl.BlockSpec((NB, dout), lambda nb: (nb, 0)),
        out_shape=jax.ShapeDtypeStruct((N, dout), jnp.float32),
    )(cnt_acc, ch_acc, x, w_root_pad)


def kernel(entity, edge_index, edge_type, edge_norm, emb, W1_rel, W1_root,
           W2_rel, W2_root):
    del edge_norm
    f32 = jnp.float32
    emb_p = jnp.pad(emb.astype(f32), ((0, 0), (0, DP - D)))
    ent_p = jnp.pad(entity.astype(jnp.int32), (0, ENT_PAD - N))
    src = edge_index[0].astype(jnp.int32)
    dst = edge_index[1].astype(jnp.int32)
    et = edge_type.astype(jnp.int32)
    src2d = jnp.pad(src, (0, EP - E)).reshape(EROWS, 128)
    et2d = jnp.pad(et, (0, EP - E)).reshape(EROWS, 128)
    dst2d = jnp.pad(dst, (0, EP - E), constant_values=N).reshape(EROWS, 128)
    w1p = jnp.pad(W1_rel.astype(f32), ((0, 0), (0, DP - D), (0, DP - D)))
    w2p = jnp.pad(W2_rel.astype(f32), ((0, 0), (0, DP - D), (0, DP - D)))
    w1rp = jnp.pad(W1_root.astype(f32), ((0, DP - D), (0, DP - D)))
    w2rp = jnp.pad(W2_root.astype(f32), ((0, DP - D), (0, DP - D)))

    ridx4 = _prep_ridx(et2d, src2d)
    x1 = _gather_rows(emb_p, ent_p)

    z1 = _rel_transform(x1[:N], w1p).reshape(R * N * NCH, CW)
    cnt_acc, ch1 = _agg_l1(z1, ridx4, dst2d)
    x2 = _combine(cnt_acc, ch1, x1[:N], w1rp, relu=True, dout=DP)

    z2 = _rel_transform(x2, w2p).reshape(R * N * NCH, CW)
    ch2 = _agg_l2(z2, ridx4, dst2d)
    return _combine(cnt_acc, ch2, x2, w2rp, relu=False, dout=D)
